# Initial kernel scaffold; baseline (speedup 1.0000x reference)
import jax
import jax.numpy as jnp
from jax.experimental import pallas as pl


def kernel(z, codebook):
    raise NotImplementedError("write your pallas kernel here")


# speedup vs baseline: 1.3063x; 1.3063x over previous
import functools

import jax
import jax.numpy as jnp
from jax import lax
from jax.experimental import pallas as pl
from jax.experimental.pallas import tpu as pltpu
from jax.experimental.pallas import tpu_sc as plsc

_NUM_CODES = 8192
_CODE_DIM = 256
_N_TOKENS = 16384
_BM = 256
_SCALE = 1.25 / (_N_TOKENS * _CODE_DIM)


def _dist_body(z_ref, cb_ref, csq_ref, idx_ref, loss_ref, acc_ref):
    i = pl.program_id(0)
    z = z_ref[...]
    cb = cb_ref[...]
    csq = csq_ref[...]
    zsq = jnp.sum(z * z, axis=1, keepdims=True)
    mm = lax.dot_general(z, cb, (((1,), (1,)), ((), ())),
                         preferred_element_type=jnp.float32)
    d = (zsq + csq) - 2.0 * mm
    dmin = jnp.min(d, axis=1, keepdims=True)
    jidx = lax.broadcasted_iota(jnp.int32, d.shape, 1)
    cand = jnp.where(d == dmin, jidx, _NUM_CODES)
    idx_ref[...] = jnp.min(cand, axis=1, keepdims=True)

    @pl.when(i == 0)
    def _():
        acc_ref[0] = 0.0

    acc_ref[0] += jnp.sum(dmin)
    loss_ref[0] = acc_ref[0] * _SCALE


def _distances_argmin(z, codebook, csq):
    grid = (_N_TOKENS // _BM,)
    return pl.pallas_call(
        _dist_body,
        grid=grid,
        in_specs=[
            pl.BlockSpec((_BM, _CODE_DIM), lambda i: (i, 0)),
            pl.BlockSpec((_NUM_CODES, _CODE_DIM), lambda i: (0, 0)),
            pl.BlockSpec((1, _NUM_CODES), lambda i: (0, 0)),
        ],
        out_specs=[
            pl.BlockSpec((_BM, 1), lambda i: (i, 0)),
            pl.BlockSpec(memory_space=pltpu.SMEM),
        ],
        out_shape=[
            jax.ShapeDtypeStruct((_N_TOKENS, 1), jnp.int32),
            jax.ShapeDtypeStruct((1,), jnp.float32),
        ],
        scratch_shapes=[pltpu.SMEM((1,), jnp.float32)],
    )(z, codebook, csq)


_N_WORKERS = 32
_B_PER_W = _N_TOKENS // _N_WORKERS
_CHUNK = 128


def _gather_body(idx_hbm, cb_hbm, out_hbm, idx_v, rows_v, sem):
    wid = lax.axis_index("s") * 2 + lax.axis_index("c")
    for c in range(_B_PER_W // _CHUNK):
        base = wid * _B_PER_W + c * _CHUNK
        pltpu.sync_copy(idx_hbm.at[pl.ds(base, _CHUNK)], idx_v)
        pltpu.async_copy(cb_hbm.at[idx_v], rows_v, sem).wait()
        pltpu.sync_copy(rows_v, out_hbm.at[pl.ds(base, _CHUNK)])


def _gather_rows(indices, codebook):
    mesh = plsc.VectorSubcoreMesh(core_axis_name="c", subcore_axis_name="s")
    gk = functools.partial(
        pl.kernel,
        mesh=mesh,
        out_type=jax.ShapeDtypeStruct((_N_TOKENS, _CODE_DIM), jnp.float32),
        scratch_types=[
            pltpu.VMEM((_CHUNK,), jnp.int32),
            pltpu.VMEM((_CHUNK, _CODE_DIM), jnp.float32),
            pltpu.SemaphoreType.DMA,
        ],
    )(_gather_body)
    return gk(indices, codebook)


def kernel(z, codebook):
    csq = jnp.sum(codebook * codebook, axis=1, keepdims=True)
    idx2, loss_v = _distances_argmin(z, codebook, csq.reshape(1, _NUM_CODES))
    indices = idx2.reshape(_N_TOKENS)
    z_q = _gather_rows(indices, codebook)
    return (z_q, indices, loss_v[0])


# speedup vs baseline: 1.5873x; 1.2150x over previous
import functools

import jax
import jax.numpy as jnp
from jax import lax
from jax.experimental import pallas as pl
from jax.experimental.pallas import tpu as pltpu
from jax.experimental.pallas import tpu_sc as plsc

_NUM_CODES = 8192
_CODE_DIM = 256
_N_TOKENS = 16384
_BM = 256
_SCALE = 1.25 / (_N_TOKENS * _CODE_DIM)


_GW = 128
_N_GROUPS = _NUM_CODES // _GW
_RS = 64


def _dist_body(z_ref, cb_ref, csq_ref, idx_ref, loss_ref, acc_ref):
    i = pl.program_id(0)
    z = z_ref[...]
    cb = cb_ref[...]
    csq = csq_ref[...]
    zsq = jnp.sum(z * z, axis=1, keepdims=True)
    mm2 = lax.dot_general(z + z, cb, (((1,), (1,)), ((), ())),
                          preferred_element_type=jnp.float32)

    lane = lax.broadcasted_iota(jnp.int32, (_RS, _GW), 1)
    part = None
    for r in range(0, _BM, _RS):
        zsq_r = zsq[r:r + _RS]

        def dist_g(g):
            s = zsq_r + csq[:, g * _GW:(g + 1) * _GW]
            return s - mm2[r:r + _RS, g * _GW:(g + 1) * _GW]

        rmin = dist_g(0)
        rgrp = jnp.zeros((_RS, _GW), jnp.int32)
        for g in range(1, _N_GROUPS):
            dg = dist_g(g)
            lt = dg < rmin
            rmin = jnp.where(lt, dg, rmin)
            rgrp = jnp.where(lt, g, rgrp)

        cidx = rgrp * _GW + lane
        dmin = jnp.min(rmin, axis=1, keepdims=True)
        cand = jnp.where(rmin == dmin, cidx, _NUM_CODES)
        idx_ref[r:r + _RS, :] = jnp.min(cand, axis=1, keepdims=True)
        ps = jnp.sum(dmin)
        part = ps if part is None else part + ps

    @pl.when(i == 0)
    def _():
        acc_ref[0] = 0.0

    acc_ref[0] += part
    loss_ref[0] = acc_ref[0] * _SCALE


def _distances_argmin(z, codebook, csq):
    grid = (_N_TOKENS // _BM,)
    return pl.pallas_call(
        _dist_body,
        grid=grid,
        in_specs=[
            pl.BlockSpec((_BM, _CODE_DIM), lambda i: (i, 0)),
            pl.BlockSpec((_NUM_CODES, _CODE_DIM), lambda i: (0, 0)),
            pl.BlockSpec((1, _NUM_CODES), lambda i: (0, 0)),
        ],
        out_specs=[
            pl.BlockSpec((_BM, 1), lambda i: (i, 0)),
            pl.BlockSpec(memory_space=pltpu.SMEM),
        ],
        out_shape=[
            jax.ShapeDtypeStruct((_N_TOKENS, 1), jnp.int32),
            jax.ShapeDtypeStruct((1,), jnp.float32),
        ],
        scratch_shapes=[pltpu.SMEM((1,), jnp.float32)],
    )(z, codebook, csq)


_N_WORKERS = 32
_B_PER_W = _N_TOKENS // _N_WORKERS
_CHUNK = 128


def _gather_body(idx_hbm, cb_hbm, out_hbm, idx_v, rows_v, sem):
    wid = lax.axis_index("s") * 2 + lax.axis_index("c")
    for c in range(_B_PER_W // _CHUNK):
        base = wid * _B_PER_W + c * _CHUNK
        pltpu.sync_copy(idx_hbm.at[pl.ds(base, _CHUNK)], idx_v)
        pltpu.async_copy(cb_hbm.at[idx_v], rows_v, sem).wait()
        pltpu.sync_copy(rows_v, out_hbm.at[pl.ds(base, _CHUNK)])


def _gather_rows(indices, codebook):
    mesh = plsc.VectorSubcoreMesh(core_axis_name="c", subcore_axis_name="s")
    gk = functools.partial(
        pl.kernel,
        mesh=mesh,
        out_type=jax.ShapeDtypeStruct((_N_TOKENS, _CODE_DIM), jnp.float32),
        scratch_types=[
            pltpu.VMEM((_CHUNK,), jnp.int32),
            pltpu.VMEM((_CHUNK, _CODE_DIM), jnp.float32),
            pltpu.SemaphoreType.DMA,
        ],
    )(_gather_body)
    return gk(indices, codebook)


def kernel(z, codebook):
    csq = jnp.sum(codebook * codebook, axis=1, keepdims=True)
    idx2, loss_v = _distances_argmin(z, codebook, csq.reshape(1, _NUM_CODES))
    indices = idx2.reshape(_N_TOKENS)
    z_q = _gather_rows(indices, codebook)
    return (z_q, indices, loss_v[0])


# speedup vs baseline: 1.9441x; 1.2248x over previous
import functools

import jax
import jax.numpy as jnp
from jax import lax
from jax.experimental import pallas as pl
from jax.experimental.pallas import tpu as pltpu
from jax.experimental.pallas import tpu_sc as plsc

_NUM_CODES = 8192
_CODE_DIM = 256
_N_TOKENS = 16384
_BM = 512
_SCALE = 1.25 / (_N_TOKENS * _CODE_DIM)
_GW = 128
_N_GROUPS = _NUM_CODES // _GW
_RS = 64


def _make_dist_body(fast):
    def body(z_ref, cb_ref, csq_ref, zsq_ref, idx_ref, loss_ref, acc_ref):
        i = pl.program_id(0)
        z = z_ref[...]
        cb = cb_ref[...]
        csq = csq_ref[...]
        mm2 = lax.dot_general(z + z, cb, (((1,), (1,)), ((), ())),
                              preferred_element_type=jnp.float32)

        lane = lax.broadcasted_iota(jnp.int32, (_RS, _GW), 1)
        part = None
        for r in range(0, _BM, _RS):
            zsq_r = zsq_ref[r:r + _RS]

            def dist_g(g):
                m = mm2[r:r + _RS, g * _GW:(g + 1) * _GW]
                if fast:
                    return zsq_r - m
                return (zsq_r + csq[:, g * _GW:(g + 1) * _GW]) - m

            rmin = dist_g(0)
            rgrp = jnp.zeros((_RS, _GW), jnp.int32)
            for g in range(1, _N_GROUPS):
                dg = dist_g(g)
                lt = dg < rmin
                rmin = jnp.where(lt, dg, rmin)
                rgrp = jnp.where(lt, g, rgrp)

            cidx = rgrp * _GW + lane
            dmin = jnp.min(rmin, axis=1, keepdims=True)
            cand = jnp.where(rmin == dmin, cidx, _NUM_CODES)
            idx_ref[r:r + _RS, :] = jnp.min(cand, axis=1, keepdims=True)
            ps = jnp.sum(dmin)
            part = ps if part is None else part + ps

        @pl.when(i == 0)
        def _():
            acc_ref[0] = 0.0

        acc_ref[0] += part
        loss_ref[0] = acc_ref[0] * _SCALE

    return body


def _dist_call(fast):
    return pl.pallas_call(
        _make_dist_body(fast),
        grid=(_N_TOKENS // _BM,),
        in_specs=[
            pl.BlockSpec((_BM, _CODE_DIM), lambda i: (i, 0)),
            pl.BlockSpec((_NUM_CODES, _CODE_DIM), lambda i: (0, 0)),
            pl.BlockSpec((1, _NUM_CODES), lambda i: (0, 0)),
            pl.BlockSpec((_BM, 1), lambda i: (i, 0)),
        ],
        out_specs=[
            pl.BlockSpec((_BM, 1), lambda i: (i, 0)),
            pl.BlockSpec(memory_space=pltpu.SMEM),
        ],
        out_shape=[
            jax.ShapeDtypeStruct((_N_TOKENS, 1), jnp.int32),
            jax.ShapeDtypeStruct((1,), jnp.float32),
        ],
        scratch_shapes=[pltpu.SMEM((1,), jnp.float32)],
    )


_N_WORKERS = 32
_B_PER_W = _N_TOKENS // _N_WORKERS
_CHUNK = 128


def _gather_body(idx_hbm, cb_hbm, out_hbm, idx_v, rows_v, sem):
    wid = lax.axis_index("s") * 2 + lax.axis_index("c")
    for c in range(_B_PER_W // _CHUNK):
        base = wid * _B_PER_W + c * _CHUNK
        pltpu.sync_copy(idx_hbm.at[pl.ds(base, _CHUNK)], idx_v)
        pltpu.async_copy(cb_hbm.at[idx_v], rows_v, sem).wait()
        pltpu.sync_copy(rows_v, out_hbm.at[pl.ds(base, _CHUNK)])


def _gather_rows(indices, codebook):
    mesh = plsc.VectorSubcoreMesh(core_axis_name="c", subcore_axis_name="s")
    gk = functools.partial(
        pl.kernel,
        mesh=mesh,
        out_type=jax.ShapeDtypeStruct((_N_TOKENS, _CODE_DIM), jnp.float32),
        scratch_types=[
            pltpu.VMEM((_CHUNK,), jnp.int32),
            pltpu.VMEM((_CHUNK, _CODE_DIM), jnp.float32),
            pltpu.SemaphoreType.DMA,
        ],
    )(_gather_body)
    return gk(indices, codebook)


def kernel(z, codebook):
    csq_col = jnp.sum(codebook * codebook, axis=1, keepdims=True)
    csq_row = csq_col.reshape(1, _NUM_CODES)
    zsq = jnp.sum(z * z, axis=1, keepdims=True)
    fast_ok = jnp.logical_and(jnp.min(zsq) >= 129.0,
                              jnp.max(csq_col) < 2.0 ** -18)
    idx2, loss_v = lax.cond(
        fast_ok,
        lambda: _dist_call(True)(z, codebook, csq_row, zsq),
        lambda: _dist_call(False)(z, codebook, csq_row, zsq),
    )
    indices = idx2.reshape(_N_TOKENS)
    z_q = _gather_rows(indices, codebook)
    return (z_q, indices, loss_v[0])
